# baseline (speedup 1.0000x reference)
import functools

import jax
import jax.numpy as jnp
from jax import lax
from jax.experimental import pallas as pl
from jax.experimental.pallas import tpu as pltpu

ROWS = 16384
COLS = 201
BLK = 2048


def _body(inv_t_ref, y_ref, o_ref):
    inv_t = inv_t_ref[0]
    y = y_ref[...] * inv_t
    e = jnp.exp(y)
    col = lax.broadcasted_iota(jnp.int32, (BLK, COLS), 1)
    m = (y > 0.0) & (col >= 1)
    ng = jnp.sum(jnp.where(m, e, 0.0), axis=1)
    o_ref[...] = jnp.log(e[:, 0] + ng) - y[:, 0]


def kernel(y_pred, mask_zeros, temperature_):
    del mask_zeros
    inv_t = (1.0 / temperature_).astype(jnp.float32)
    grid = (ROWS // BLK,)
    out = pl.pallas_call(
        _body,
        grid=grid,
        in_specs=[
            pl.BlockSpec(memory_space=pltpu.SMEM),
            pl.BlockSpec((BLK, COLS), lambda i: (i, 0)),
        ],
        out_specs=pl.BlockSpec((BLK,), lambda i: (i,)),
        out_shape=jax.ShapeDtypeStruct((ROWS,), jnp.float32),
    )(inv_t, y_pred)
    return (out, 0.0)
